# baseline (speedup 1.0000x reference)
import jax
import jax.numpy as jnp
from jax import lax
from jax.experimental import pallas as pl
from jax.experimental.pallas import tpu as pltpu
from jax.experimental.pallas import tpu_sc as plsc

B = 4096
F = 26
L = 50
V = 100000
D = 64
NDENSE = 4

NC = 2
NS = 16
NW = NC * NS
SAMP = B // NW


def _body(sidx_t, lidx_t, tabf, ltab, sp_out, pool_out,
          idx_v, rows_v, lidx_v, acc_v, pooled_v, row0_v, n0_v, inv_v, sem):
    wid = lax.axis_index("s") * NC + lax.axis_index("c")
    base = wid * SAMP

    pltpu.sync_copy(lidx_t.at[:, pl.ds(base, SAMP)], lidx_v)
    pltpu.sync_copy(ltab.at[pl.ds(0, 1)], row0_v)

    pltpu.async_copy(ltab.at[lidx_v.at[0]], acc_v, sem).wait()

    @pl.loop(1, L)
    def _pool(j):
        pltpu.async_copy(ltab.at[lidx_v.at[j]], acc_v, sem, add=True).wait()

    for sg in range(SAMP // 16):
        sl = pl.ds(sg * 16, 16)

        def _cnt(j, c):
            z = lidx_v[j, sl] == 0
            return c + jnp.where(z, jnp.float32(1.0), jnp.float32(0.0))

        cnt = lax.fori_loop(0, L, _cnt, jnp.zeros((16,), jnp.float32))
        n0_v[sl] = cnt
        inv_v[sl] = jnp.float32(1.0) / jnp.maximum(
            jnp.float32(L) - cnt, jnp.float32(1.0))

    @pl.loop(0, SAMP // 16)
    def _fin(sg):
        s0 = sg * 16
        n0vec = n0_v[pl.ds(s0, 16)]
        invvec = inv_v[pl.ds(s0, 16)]
        for k in range(16):
            n0s = n0vec[k]
            invs = invvec[k]
            for dd in range(D // 16):
                sl = pl.ds(dd * 16, 16)
                pooled_v[s0 + k, sl] = (
                    acc_v[s0 + k, sl] - n0s * row0_v[0, sl]) * invs

    pltpu.sync_copy(pooled_v, pool_out.at[pl.ds(base, SAMP), :])

    @pl.loop(0, F)
    def _field(f):
        pltpu.sync_copy(sidx_t.at[pl.ds(f, 1), pl.ds(base, SAMP)], idx_v)
        off = f * (V + 1)
        for g in range(SAMP // 16):
            sl = pl.ds(g * 16, 16)
            idx_v[0, sl] = idx_v[0, sl] + off
        pltpu.async_copy(tabf.at[idx_v.at[0]], rows_v, sem).wait()
        pltpu.sync_copy(
            rows_v, sp_out.at[pl.ds(base, SAMP), pl.ds(f * D, D)])


@jax.jit
def kernel(sparse_idx, list_idx, dense_vals, sparse_tables, list_table):
    sidx_t = sparse_idx.T
    lidx_t = list_idx.T
    tabf = sparse_tables.reshape(F * (V + 1), D)

    mesh = plsc.VectorSubcoreMesh(core_axis_name="c", subcore_axis_name="s")
    run = pl.kernel(
        _body,
        out_type=(
            jax.ShapeDtypeStruct((B, F * D), jnp.float32),
            jax.ShapeDtypeStruct((B, D), jnp.float32),
        ),
        mesh=mesh,
        compiler_params=pltpu.CompilerParams(use_tc_tiling_on_sc=False),
        scratch_types=[
            pltpu.VMEM((1, SAMP), jnp.int32),
            pltpu.VMEM((SAMP, D), jnp.float32),
            pltpu.VMEM((L, SAMP), jnp.int32),
            pltpu.VMEM((SAMP, D), jnp.float32),
            pltpu.VMEM((SAMP, D), jnp.float32),
            pltpu.VMEM((1, D), jnp.float32),
            pltpu.VMEM((SAMP,), jnp.float32),
            pltpu.VMEM((SAMP,), jnp.float32),
            pltpu.SemaphoreType.DMA,
        ],
    )
    sp_out, pool_out = run(sidx_t, lidx_t, tabf, list_table)
    return jnp.concatenate([dense_vals, sp_out, pool_out], axis=-1)


# speedup vs baseline: 1.0078x; 1.0078x over previous
import jax
import jax.numpy as jnp
from jax import lax
from jax.experimental import pallas as pl
from jax.experimental.pallas import tpu as pltpu
from jax.experimental.pallas import tpu_sc as plsc

B = 4096
F = 26
L = 50
V = 100000
D = 64
NDENSE = 4
DOUT = NDENSE + F * D + D

NC = 2
NS = 16
NW = NC * NS
SAMP = B // NW
CH = 8
NCHUNK = SAMP // CH


def _body(sidx, lidx, dense1d, tabf, ltab, out,
          sraw, lraw, lidx_t, dv, acc, row0, n0_v, inv_v,
          asm0, asm1, sf0, sf1,
          sem_m, sem_p, sem_g0, sem_g1, sem_w0, sem_w1):
    wid = lax.axis_index("s") * NC + lax.axis_index("c")
    base = wid * SAMP
    asms = (asm0, asm1)
    sfs = (sf0, sf1)
    sem_g = (sem_g0, sem_g1)
    sem_w = (sem_w0, sem_w1)
    iota = lax.iota(jnp.int32, 16)

    cp1 = pltpu.async_copy(sidx.at[pl.ds(base, SAMP), :], sraw, sem_m)
    cp2 = pltpu.async_copy(lidx.at[pl.ds(base, SAMP), :], lraw, sem_m)
    cp3 = pltpu.async_copy(dense1d.at[pl.ds(base * NDENSE, SAMP * NDENSE)],
                           dv, sem_m)
    cp4 = pltpu.async_copy(ltab.at[pl.ds(0, 1)], row0, sem_m)
    cp1.wait()
    cp2.wait()
    cp3.wait()
    cp4.wait()

    zero16 = jnp.zeros((16,), jnp.float32)

    @pl.loop(0, SAMP)
    def _zero(s):
        for dd in range(D // 16):
            acc[s, pl.ds(dd * 16, 16)] = zero16

    @pl.loop(0, L)
    def _tr(j):
        jcol = jnp.full((16,), 0, jnp.int32) + j
        for g in range(SAMP // 16):
            rows = g * 16 + iota
            v = plsc.load_gather(lraw, [rows, jcol])
            lidx_t[j, pl.ds(g * 16, 16)] = v

    @pl.loop(0, L)
    def _pool(j):
        pltpu.async_copy(ltab.at[lidx_t.at[j]], acc, sem_p, add=True)

    off_a = iota * (V + 1)
    off_b = jnp.where(iota >= 6, (iota + 10) * (V + 1), 0)

    @pl.loop(0, SAMP)
    def _off(s):
        sraw[s, pl.ds(0, 16)] = sraw[s, pl.ds(0, 16)] + off_a
        sraw[s, pl.ds(10, 16)] = sraw[s, pl.ds(10, 16)] + off_b

    def fire_gathers(c, par):
        buf = sfs[par]
        sem = sem_g[par]

        @pl.loop(0, CH)
        def _g(k):
            s = c * CH + k
            pltpu.async_copy(tabf.at[sraw.at[s]], buf.at[k], sem)

    def drain_gathers(par):
        buf = sfs[par]
        sem = sem_g[par]

        @pl.loop(0, CH)
        def _d(k):
            pltpu.make_async_copy(tabf.at[sraw.at[0]], buf.at[0], sem).wait()

    def write_desc(c, par):
        return pltpu.make_async_copy(
            asms[par], out.at[pl.ds(base + c * CH, CH), :], sem_w[par])

    fire_gathers(0, 0)

    for sg in range(SAMP // 16):
        sl = pl.ds(sg * 16, 16)

        def _cnt(j, c):
            z = lidx_t[j, sl] == 0
            return c + jnp.where(z, jnp.float32(1.0), jnp.float32(0.0))

        cnt = lax.fori_loop(0, L, _cnt, jnp.zeros((16,), jnp.float32))
        n0_v[sl] = cnt
        inv_v[sl] = jnp.float32(1.0) / jnp.maximum(
            jnp.float32(L) - cnt, jnp.float32(1.0))

    @pl.loop(0, L)
    def _pdrain(j):
        pltpu.make_async_copy(ltab.at[lidx_t.at[0]], acc, sem_p).wait()

    def assemble_and_write(c, t, par):
        buf = asms[par]
        sbuf = sfs[par]
        s0 = c * CH

        @pl.loop(0, CH)
        def _asm(k):
            for f in range(F):
                for dd in range(D // 16):
                    buf[k, pl.ds(NDENSE + f * D + dd * 16, 16)] = (
                        sbuf[k, f, pl.ds(dd * 16, 16)])

        n0vec = n0_v[pl.ds(t * 16, 16)]
        invvec = inv_v[pl.ds(t * 16, 16)]
        for k in range(CH):
            n0s = n0vec[par * CH + k]
            invs = invvec[par * CH + k]
            for dd in range(D // 16):
                sl = pl.ds(dd * 16, 16)
                buf[k, pl.ds(NDENSE + F * D + dd * 16, 16)] = (
                    acc[s0 + k, sl] - n0s * row0[0, sl]) * invs
        for g in range(CH * NDENSE // 16):
            vals = dv[pl.ds(c * (CH * NDENSE) + g * 16, 16)]
            rows = g * 4 + iota // 4
            cols = iota % 4
            plsc.store_scatter(buf, [rows, cols], vals)
        pltpu.async_copy(buf, out.at[pl.ds(base + s0, CH), :], sem_w[par])

    @pl.loop(0, NCHUNK // 2)
    def _chunks(t):
        fire_gathers(2 * t + 1, 1)
        drain_gathers(0)

        @pl.when(t >= 1)
        def _dw0():
            write_desc(2 * t - 2, 0).wait()

        assemble_and_write(2 * t, t, 0)

        @pl.when(t <= NCHUNK // 2 - 2)
        def _fg0():
            fire_gathers(2 * t + 2, 0)

        drain_gathers(1)

        @pl.when(t >= 1)
        def _dw1():
            write_desc(2 * t - 1, 1).wait()

        assemble_and_write(2 * t + 1, t, 1)

    write_desc(NCHUNK - 2, 0).wait()
    write_desc(NCHUNK - 1, 1).wait()


@jax.jit
def kernel(sparse_idx, list_idx, dense_vals, sparse_tables, list_table):
    tabf = sparse_tables.reshape(F * (V + 1), D)
    dense1d = dense_vals.reshape(B * NDENSE)

    mesh = plsc.VectorSubcoreMesh(core_axis_name="c", subcore_axis_name="s")
    run = pl.kernel(
        _body,
        out_type=jax.ShapeDtypeStruct((B, DOUT), jnp.float32),
        mesh=mesh,
        compiler_params=pltpu.CompilerParams(
            use_tc_tiling_on_sc=False, needs_layout_passes=False),
        scratch_types=[
            pltpu.VMEM((SAMP, F), jnp.int32),
            pltpu.VMEM((SAMP, L), jnp.int32),
            pltpu.VMEM((L, SAMP), jnp.int32),
            pltpu.VMEM((SAMP * NDENSE,), jnp.float32),
            pltpu.VMEM((SAMP, D), jnp.float32),
            pltpu.VMEM((1, D), jnp.float32),
            pltpu.VMEM((SAMP,), jnp.float32),
            pltpu.VMEM((SAMP,), jnp.float32),
            pltpu.VMEM((CH, DOUT), jnp.float32),
            pltpu.VMEM((CH, DOUT), jnp.float32),
            pltpu.VMEM((CH, F, D), jnp.float32),
            pltpu.VMEM((CH, F, D), jnp.float32),
            pltpu.SemaphoreType.DMA,
            pltpu.SemaphoreType.DMA,
            pltpu.SemaphoreType.DMA,
            pltpu.SemaphoreType.DMA,
            pltpu.SemaphoreType.DMA,
            pltpu.SemaphoreType.DMA,
        ],
    )
    return run(sparse_idx, list_idx, dense1d, tabf, list_table)
